# baseline (speedup 1.0000x reference)
import jax
import jax.numpy as jnp
from jax.experimental import pallas as pl

N = 10000


def _sage_pool_conv(h, src, dst, W_pool, b_pool, W_self, W_neigh, bias):
    m = jax.nn.relu(h @ W_pool.T + b_pool)
    msgs = jnp.take(m, src, axis=0)
    pooled = jax.ops.segment_max(msgs, dst, num_segments=N)
    pooled = jnp.where(jnp.isneginf(pooled), 0.0, pooled)
    return h @ W_self.T + pooled @ W_neigh.T + bias


def _norm_relu_kernel(h_ref, o_ref):
    h = h_ref[...]
    n = jnp.sqrt(jnp.sum(h * h, axis=1, keepdims=True))
    o_ref[...] = jax.nn.relu(h / jnp.maximum(n, 1e-12))


def _norm_relu(h):
    return pl.pallas_call(
        _norm_relu_kernel,
        out_shape=jax.ShapeDtypeStruct(h.shape, h.dtype),
    )(h)


def kernel(inputs, edge_index, W_pool1, b_pool1, W_self1, W_neigh1, bias1,
           W_pool2, b_pool2, W_self2, W_neigh2, bias2):
    src = edge_index[0]
    dst = edge_index[1]
    h = _sage_pool_conv(inputs, src, dst, W_pool1, b_pool1, W_self1, W_neigh1, bias1)
    h = _norm_relu(h)
    h = _sage_pool_conv(h, src, dst, W_pool2, b_pool2, W_self2, W_neigh2, bias2)
    h = _norm_relu(h)
    return h


# speedup vs baseline: 1.0470x; 1.0470x over previous
import jax
import jax.numpy as jnp
from jax import lax
from jax.experimental import pallas as pl
from jax.experimental.pallas import tpu as pltpu
from jax.experimental.pallas import tpu_sc as plsc

N = 10000
E = 320000
D = 128

NC, NS = 2, 16
NW = NC * NS
RPW = 320
LAST = N - (NW - 1) * RPW
CHUNK = 8000
NCHUNK = E // CHUNK
KB = 64
MBUF = CHUNK + KB + 16
TRASH = CHUNK + KB
DUMMY = RPW


def _segmax_body(m_hbm, src_hbm, dst_hbm, pooled_hbm,
                 dst_buf, src_buf, mdst, msrc, idx_blk, rows, acc, sem):
    c = lax.axis_index("c")
    s = lax.axis_index("s")
    wid = s * NC + c
    lo = wid * RPW
    zf16 = jnp.zeros((16,), jnp.float32)
    zi16 = jnp.zeros((16,), jnp.int32)
    onev = jnp.ones((16,), jnp.int32)
    dummyv = jnp.full((16,), DUMMY, jnp.int32)
    trashv = jnp.full((16,), TRASH, jnp.int32)
    lov = lax.broadcast(lo, (16,))
    hiv = lax.broadcast(lo + RPW, (16,))

    def z_acc(i, _):
        acc[pl.ds(i * 16, 16)] = zf16
        return 0
    lax.fori_loop(0, (RPW + 1) * D // 16, z_acc, 0)

    def z_msrc(i, _):
        msrc[pl.ds(i * 16, 16)] = zi16
        return 0
    lax.fori_loop(0, MBUF // 16, z_msrc, 0)

    def chunk_body(ci, _):
        base = ci * CHUNK
        pltpu.sync_copy(dst_hbm.at[pl.ds(base, CHUNK)], dst_buf)
        pltpu.sync_copy(src_hbm.at[pl.ds(base, CHUNK)], src_buf)

        def scan_body(i, cnt):
            vd = dst_buf[pl.ds(i * 16, 16)]
            vs = src_buf[pl.ds(i * 16, 16)]
            msk = (vd >= lov) & (vd < hiv)
            cs = plsc.cumsum(jnp.where(msk, onev, zi16))
            cntv = lax.broadcast(cnt, (16,))
            pos = jnp.where(msk, cntv + cs - onev, trashv)
            plsc.store_scatter(mdst, [pos], vd - lov)
            plsc.store_scatter(msrc, [pos], vs)
            return cnt + cs[15]
        cnt = lax.fori_loop(0, CHUNK // 16, scan_body, 0)

        for t in range(KB // 16):
            mdst[pl.ds(cnt + t * 16, 16)] = dummyv
            msrc[pl.ds(cnt + t * 16, 16)] = zi16

        nblk = (cnt + KB - 1) // KB

        def blk_body(b, _):
            cp = pltpu.async_copy(m_hbm.at[msrc.at[pl.ds(b * KB, KB)]], rows, sem)
            cp.wait()
            for g in range(KB // 16):
                vd = mdst[pl.ds(b * KB + g * 16, 16)]
                for l in range(16):
                    rb = vd[l] * D
                    for j in range(8):
                        sl = pl.ds(rb + j * 16, 16)
                        acc[sl] = jnp.maximum(acc[sl],
                                              rows[g * 16 + l, pl.ds(j * 16, 16)])
            return 0
        lax.fori_loop(0, nblk, blk_body, 0)
        return 0
    lax.fori_loop(0, NCHUNK, chunk_body, 0)

    @pl.when(wid < NW - 1)
    def _():
        pltpu.sync_copy(acc.at[pl.ds(0, RPW * D)],
                        pooled_hbm.at[pl.ds(lo * D, RPW * D)])

    @pl.when(wid == NW - 1)
    def _():
        pltpu.sync_copy(acc.at[pl.ds(0, LAST * D)],
                        pooled_hbm.at[pl.ds(lo * D, LAST * D)])


_segmax = pl.kernel(
    _segmax_body,
    out_type=jax.ShapeDtypeStruct((N * D,), jnp.float32),
    mesh=plsc.VectorSubcoreMesh(core_axis_name="c", subcore_axis_name="s"),
    scratch_types=[
        pltpu.VMEM((CHUNK,), jnp.int32),
        pltpu.VMEM((CHUNK,), jnp.int32),
        pltpu.VMEM((MBUF,), jnp.int32),
        pltpu.VMEM((MBUF,), jnp.int32),
        pltpu.VMEM((KB,), jnp.int32),
        pltpu.VMEM((KB, D), jnp.float32),
        pltpu.VMEM(((RPW + 1) * D,), jnp.float32),
        pltpu.SemaphoreType.DMA,
    ],
    compiler_params=pltpu.CompilerParams(needs_layout_passes=False),
)


def _dot_t(x, w):
    return lax.dot_general(x, w, (((1,), (1,)), ((), ())),
                           preferred_element_type=jnp.float32)


def _pre_body(h_ref, w_ref, b_ref, o_ref):
    m = _dot_t(h_ref[...], w_ref[...]) + b_ref[...]
    o_ref[...] = jnp.maximum(m, 0.0)


def _pre(h, w, b):
    return pl.pallas_call(
        _pre_body,
        out_shape=jax.ShapeDtypeStruct((h.shape[0], w.shape[0]), jnp.float32),
    )(h, w, b.reshape(1, -1))


def _post_body(h_ref, p_ref, ws_ref, wn_ref, b_ref, o_ref):
    t = _dot_t(h_ref[...], ws_ref[...]) + _dot_t(p_ref[...], wn_ref[...])
    t = t + b_ref[...]
    nrm = jnp.sqrt(jnp.sum(t * t, axis=1, keepdims=True))
    t = t / jnp.maximum(nrm, 1e-12)
    o_ref[...] = jnp.maximum(t, 0.0)


def _post(h, pooled, ws, wn, b):
    return pl.pallas_call(
        _post_body,
        out_shape=jax.ShapeDtypeStruct((h.shape[0], ws.shape[0]), jnp.float32),
    )(h, pooled, ws, wn, b.reshape(1, -1))


def kernel(inputs, edge_index, W_pool1, b_pool1, W_self1, W_neigh1, bias1,
           W_pool2, b_pool2, W_self2, W_neigh2, bias2):
    src = edge_index[0]
    dst = edge_index[1]
    m1 = _pre(inputs, W_pool1, b_pool1)
    pooled1 = _segmax(m1, src, dst).reshape(N, D)
    h1 = _post(inputs, pooled1, W_self1, W_neigh1, bias1)
    m2 = _pre(h1, W_pool2, b_pool2)
    pooled2 = _segmax(m2, src, dst).reshape(N, D)
    return _post(h1, pooled2, W_self2, W_neigh2, bias2)


# speedup vs baseline: 1.7577x; 1.6788x over previous
import jax
import jax.numpy as jnp
from jax import lax
from jax.experimental import pallas as pl
from jax.experimental.pallas import tpu as pltpu
from jax.experimental.pallas import tpu_sc as plsc

N = 10000
E = 320000
D = 128

NC, NS = 2, 16
NW = NC * NS
RPW = 320
LAST = N - (NW - 1) * RPW
CHUNK = 8000
NCHUNK = E // CHUNK
KB = 32
MBUF = CHUNK + KB + 16
TRASH = CHUNK + KB
DUMMY = RPW


def _segmax_body(m_hbm, src_hbm, dst_hbm, pooled_hbm,
                 dst0, src0, dst1, src1, mdst, msrc, rows0, rows1, acc,
                 sd0, ss0, sd1, ss1, sg0, sg1):
    c = lax.axis_index("c")
    s = lax.axis_index("s")
    wid = s * NC + c
    lo = wid * RPW
    zf16 = jnp.zeros((16,), jnp.float32)
    zi16 = jnp.zeros((16,), jnp.int32)
    onev = jnp.ones((16,), jnp.int32)
    dummyv = jnp.full((16,), DUMMY, jnp.int32)
    trashv = jnp.full((16,), TRASH, jnp.int32)
    lov = lax.broadcast(lo, (16,))
    hiv = lax.broadcast(lo + RPW, (16,))

    def z_acc(i, _):
        acc[pl.ds(i * 16, 16)] = zf16
        return 0
    lax.fori_loop(0, (RPW + 1) * D // 16, z_acc, 0)

    def z_msrc(i, _):
        msrc[pl.ds(i * 16, 16)] = zi16
        return 0
    lax.fori_loop(0, MBUF // 16, z_msrc, 0)

    def stage(ci, db, sb, semd, sems):
        base = ci * CHUNK
        cpd = pltpu.make_async_copy(dst_hbm.at[pl.ds(base, CHUNK)], db, semd)
        cps = pltpu.make_async_copy(src_hbm.at[pl.ds(base, CHUNK)], sb, sems)
        cpd.start()
        cps.start()
        return cpd, cps

    def gather_blk(b, rbuf, sem):
        return pltpu.make_async_copy(
            m_hbm.at[msrc.at[pl.ds(b * KB, KB)]], rbuf, sem)

    def process_chunk(db, sb):
        def scan_body(i, cntv):
            vd = db[pl.ds(i * 16, 16)]
            vs = sb[pl.ds(i * 16, 16)]
            msk = (vd >= lov) & (vd < hiv)
            cs = plsc.cumsum(jnp.where(msk, onev, zi16))
            pos = jnp.where(msk, cntv + cs - onev, trashv)
            plsc.store_scatter(mdst, [pos], vd - lov)
            plsc.store_scatter(msrc, [pos], vs)
            n = plsc.all_reduce_population_count(msk)
            return cntv + n
        cntv = lax.fori_loop(0, CHUNK // 16, scan_body, zi16)
        cnt = cntv[0]

        for t in range(KB // 16):
            mdst[pl.ds(cnt + t * 16, 16)] = dummyv
            msrc[pl.ds(cnt + t * 16, 16)] = zi16

        nblk = (cnt + KB - 1) // KB

        def process_blk(b, rbuf):
            for g in range(KB // 16):
                vd = mdst[pl.ds(b * KB + g * 16, 16)]
                for l in range(16):
                    rb = vd[l] * D
                    for j in range(8):
                        sl = pl.ds(rb + j * 16, 16)
                        acc[sl] = jnp.maximum(
                            acc[sl], rbuf[g * 16 + l, pl.ds(j * 16, 16)])

        @pl.when(nblk > 0)
        def _():
            gather_blk(0, rows0, sg0).start()

        def pair_body(p, _):
            b0 = p * 2
            b1 = b0 + 1

            @pl.when(b1 < nblk)
            def _():
                gather_blk(b1, rows1, sg1).start()

            gather_blk(b0, rows0, sg0).wait()
            process_blk(b0, rows0)

            @pl.when(b0 + 2 < nblk)
            def _():
                gather_blk(b0 + 2, rows0, sg0).start()

            @pl.when(b1 < nblk)
            def _():
                gather_blk(b1, rows1, sg1).wait()
                process_blk(b1, rows1)
            return 0
        lax.fori_loop(0, (nblk + 1) // 2, pair_body, 0)

    stage(0, dst0, src0, sd0, ss0)

    def cpair_body(p, _):
        c0 = p * 2
        c1 = c0 + 1
        stage(c1, dst1, src1, sd1, ss1)
        pltpu.make_async_copy(dst_hbm.at[pl.ds(c0 * CHUNK, CHUNK)], dst0, sd0).wait()
        pltpu.make_async_copy(src_hbm.at[pl.ds(c0 * CHUNK, CHUNK)], src0, ss0).wait()
        process_chunk(dst0, src0)

        @pl.when(c0 + 2 < NCHUNK)
        def _():
            stage(c0 + 2, dst0, src0, sd0, ss0)

        pltpu.make_async_copy(dst_hbm.at[pl.ds(c1 * CHUNK, CHUNK)], dst1, sd1).wait()
        pltpu.make_async_copy(src_hbm.at[pl.ds(c1 * CHUNK, CHUNK)], src1, ss1).wait()
        process_chunk(dst1, src1)
        return 0
    lax.fori_loop(0, NCHUNK // 2, cpair_body, 0)

    @pl.when(wid < NW - 1)
    def _():
        pltpu.sync_copy(acc.at[pl.ds(0, RPW * D)],
                        pooled_hbm.at[pl.ds(lo * D, RPW * D)])

    @pl.when(wid == NW - 1)
    def _():
        pltpu.sync_copy(acc.at[pl.ds(0, LAST * D)],
                        pooled_hbm.at[pl.ds(lo * D, LAST * D)])


_segmax = pl.kernel(
    _segmax_body,
    out_type=jax.ShapeDtypeStruct((N * D,), jnp.float32),
    mesh=plsc.VectorSubcoreMesh(core_axis_name="c", subcore_axis_name="s"),
    scratch_types=[
        pltpu.VMEM((CHUNK,), jnp.int32),
        pltpu.VMEM((CHUNK,), jnp.int32),
        pltpu.VMEM((CHUNK,), jnp.int32),
        pltpu.VMEM((CHUNK,), jnp.int32),
        pltpu.VMEM((MBUF,), jnp.int32),
        pltpu.VMEM((MBUF,), jnp.int32),
        pltpu.VMEM((KB, D), jnp.float32),
        pltpu.VMEM((KB, D), jnp.float32),
        pltpu.VMEM(((RPW + 1) * D,), jnp.float32),
        pltpu.SemaphoreType.DMA,
        pltpu.SemaphoreType.DMA,
        pltpu.SemaphoreType.DMA,
        pltpu.SemaphoreType.DMA,
        pltpu.SemaphoreType.DMA,
        pltpu.SemaphoreType.DMA,
    ],
    compiler_params=pltpu.CompilerParams(needs_layout_passes=False),
)


def _dot_t(x, w):
    return lax.dot_general(x, w, (((1,), (1,)), ((), ())),
                           preferred_element_type=jnp.float32)


def _pre_body(h_ref, w_ref, b_ref, o_ref):
    m = _dot_t(h_ref[...], w_ref[...]) + b_ref[...]
    o_ref[...] = jnp.maximum(m, 0.0)


def _pre(h, w, b):
    return pl.pallas_call(
        _pre_body,
        out_shape=jax.ShapeDtypeStruct((h.shape[0], w.shape[0]), jnp.float32),
    )(h, w, b.reshape(1, -1))


def _post_body(h_ref, p_ref, ws_ref, wn_ref, b_ref, o_ref):
    t = _dot_t(h_ref[...], ws_ref[...]) + _dot_t(p_ref[...], wn_ref[...])
    t = t + b_ref[...]
    nrm = jnp.sqrt(jnp.sum(t * t, axis=1, keepdims=True))
    t = t / jnp.maximum(nrm, 1e-12)
    o_ref[...] = jnp.maximum(t, 0.0)


def _post(h, pooled, ws, wn, b):
    return pl.pallas_call(
        _post_body,
        out_shape=jax.ShapeDtypeStruct((h.shape[0], ws.shape[0]), jnp.float32),
    )(h, pooled, ws, wn, b.reshape(1, -1))


def kernel(inputs, edge_index, W_pool1, b_pool1, W_self1, W_neigh1, bias1,
           W_pool2, b_pool2, W_self2, W_neigh2, bias2):
    src = edge_index[0]
    dst = edge_index[1]
    m1 = _pre(inputs, W_pool1, b_pool1)
    pooled1 = _segmax(m1, src, dst).reshape(N, D)
    h1 = _post(inputs, pooled1, W_self1, W_neigh1, bias1)
    m2 = _pre(h1, W_pool2, b_pool2)
    pooled2 = _segmax(m2, src, dst).reshape(N, D)
    return _post(h1, pooled2, W_self2, W_neigh2, bias2)


# speedup vs baseline: 1.7625x; 1.0027x over previous
import jax
import jax.numpy as jnp
from jax import lax
from jax.experimental import pallas as pl
from jax.experimental.pallas import tpu as pltpu
from jax.experimental.pallas import tpu_sc as plsc

N = 10000
E = 320000
D = 128

NC, NS = 2, 16
NW = NC * NS
RPW = 320
LAST = N - (NW - 1) * RPW
CHUNK = 8000
NCHUNK = E // CHUNK
KB = 32
MBUF = CHUNK + KB + 16
TRASH = CHUNK + KB
DUMMY = RPW
FB = 8192
FCAP = 16384
BQ = 512
BCAP = E + NCHUNK * 16 + BQ
BCAP = (BCAP + BQ - 1) // BQ * BQ


def _partition_body(src_hbm, dst_hbm, bdst_hbm, bsrc_hbm, cnt_hbm,
                    dst0, src0, dst1, src1, mdst, msrc, fdst, fsrc, cbuf,
                    sd0, ss0, sd1, ss1, sf0, sf1):
    c = lax.axis_index("c")
    s = lax.axis_index("s")
    wid = s * NC + c
    lo = wid * RPW
    zi16 = jnp.zeros((16,), jnp.int32)
    onev = jnp.ones((16,), jnp.int32)
    dummyv = jnp.full((16,), DUMMY, jnp.int32)
    trashv = jnp.full((16,), TRASH, jnp.int32)
    lov = lax.broadcast(lo, (16,))
    hiv = lax.broadcast(lo + RPW, (16,))
    wbase = wid * BCAP

    def stage(ci, db, sb, semd, sems):
        base = ci * CHUNK
        cpd = pltpu.make_async_copy(dst_hbm.at[pl.ds(base, CHUNK)], db, semd)
        cps = pltpu.make_async_copy(src_hbm.at[pl.ds(base, CHUNK)], sb, sems)
        cpd.start()
        cps.start()
        return cpd, cps

    def process_chunk(db, sb, carry):
        fill, off = carry
        def scan_body(i, cntv):
            vd = db[pl.ds(i * 16, 16)]
            vs = sb[pl.ds(i * 16, 16)]
            msk = (vd >= lov) & (vd < hiv)
            cs = plsc.cumsum(jnp.where(msk, onev, zi16))
            pos = jnp.where(msk, cntv + cs - onev, trashv)
            plsc.store_scatter(mdst, [pos], vd - lov)
            plsc.store_scatter(msrc, [pos], vs)
            n = plsc.all_reduce_population_count(msk)
            return cntv + n
        cntv = lax.fori_loop(0, CHUNK // 16, scan_body, zi16)
        cnt = cntv[0]

        mdst[pl.ds(cnt, 16)] = dummyv
        msrc[pl.ds(cnt, 16)] = zi16
        cnt16 = (cnt + 15) // 16 * 16

        def app_body(i, _):
            fdst[pl.ds(fill + i * 16, 16)] = mdst[pl.ds(i * 16, 16)]
            fsrc[pl.ds(fill + i * 16, 16)] = msrc[pl.ds(i * 16, 16)]
            return 0
        lax.fori_loop(0, cnt16 // 16, app_body, 0)
        nfill = fill + cnt16

        @pl.when(nfill >= FB)
        def _():
            dpos = wbase + off * FB
            pltpu.sync_copy(fdst.at[pl.ds(0, FB)], bdst_hbm.at[pl.ds(dpos, FB)])
            pltpu.sync_copy(fsrc.at[pl.ds(0, FB)], bsrc_hbm.at[pl.ds(dpos, FB)])
            rem = nfill - FB

            def mv_body(i, _):
                fdst[pl.ds(i * 16, 16)] = fdst[pl.ds(FB + i * 16, 16)]
                fsrc[pl.ds(i * 16, 16)] = fsrc[pl.ds(FB + i * 16, 16)]
                return 0
            lax.fori_loop(0, (rem + 15) // 16, mv_body, 0)

        flushed = nfill >= FB
        return (jnp.where(flushed, nfill - FB, nfill),
                jnp.where(flushed, off + 1, off))

    stage(0, dst0, src0, sd0, ss0)

    def cpair_body(p, carry):
        c0 = p * 2
        c1 = c0 + 1
        stage(c1, dst1, src1, sd1, ss1)
        pltpu.make_async_copy(dst_hbm.at[pl.ds(c0 * CHUNK, CHUNK)], dst0, sd0).wait()
        pltpu.make_async_copy(src_hbm.at[pl.ds(c0 * CHUNK, CHUNK)], src0, ss0).wait()
        carry = process_chunk(dst0, src0, carry)

        @pl.when(c0 + 2 < NCHUNK)
        def _():
            stage(c0 + 2, dst0, src0, sd0, ss0)

        pltpu.make_async_copy(dst_hbm.at[pl.ds(c1 * CHUNK, CHUNK)], dst1, sd1).wait()
        pltpu.make_async_copy(src_hbm.at[pl.ds(c1 * CHUNK, CHUNK)], src1, ss1).wait()
        return process_chunk(dst1, src1, carry)

    fill, off = lax.fori_loop(0, NCHUNK // 2, cpair_body,
                              (jnp.int32(0), jnp.int32(0)))

    padv = (BQ - fill % BQ) % BQ

    def pad_body(i, _):
        fdst[pl.ds(fill + i * 16, 16)] = dummyv
        fsrc[pl.ds(fill + i * 16, 16)] = zi16
        return 0
    lax.fori_loop(0, padv // 16, pad_body, 0)
    fill = fill + padv

    def fin_body(j, _):
        dpos = wbase + off * FB + j * BQ
        pltpu.sync_copy(fdst.at[pl.ds(j * BQ, BQ)], bdst_hbm.at[pl.ds(dpos, BQ)])
        pltpu.sync_copy(fsrc.at[pl.ds(j * BQ, BQ)], bsrc_hbm.at[pl.ds(dpos, BQ)])
        return 0
    lax.fori_loop(0, fill // BQ, fin_body, 0)

    total = off * FB + fill
    cbuf[pl.ds(0, 16)] = lax.broadcast(total, (16,))
    pltpu.sync_copy(cbuf, cnt_hbm.at[pl.ds(wid * 16, 16)])


_partition = pl.kernel(
    _partition_body,
    out_type=(jax.ShapeDtypeStruct((NW * BCAP,), jnp.int32),
              jax.ShapeDtypeStruct((NW * BCAP,), jnp.int32),
              jax.ShapeDtypeStruct((NW * 16,), jnp.int32)),
    mesh=plsc.VectorSubcoreMesh(core_axis_name="c", subcore_axis_name="s"),
    scratch_types=[
        pltpu.VMEM((CHUNK,), jnp.int32),
        pltpu.VMEM((CHUNK,), jnp.int32),
        pltpu.VMEM((CHUNK,), jnp.int32),
        pltpu.VMEM((CHUNK,), jnp.int32),
        pltpu.VMEM((MBUF,), jnp.int32),
        pltpu.VMEM((MBUF,), jnp.int32),
        pltpu.VMEM((FCAP,), jnp.int32),
        pltpu.VMEM((FCAP,), jnp.int32),
        pltpu.VMEM((16,), jnp.int32),
        pltpu.SemaphoreType.DMA,
        pltpu.SemaphoreType.DMA,
        pltpu.SemaphoreType.DMA,
        pltpu.SemaphoreType.DMA,
        pltpu.SemaphoreType.DMA,
        pltpu.SemaphoreType.DMA,
    ],
    compiler_params=pltpu.CompilerParams(needs_layout_passes=False),
)


def _gathermax_body(m_hbm, bdst_hbm, bsrc_hbm, cnt_hbm, pooled_hbm,
                    d0, s0, d1, s1, rows0, rows1, acc, cbuf,
                    sed0, ses0, sed1, ses1, sg0, sg1):
    c = lax.axis_index("c")
    s = lax.axis_index("s")
    wid = s * NC + c
    lo = wid * RPW
    zf16 = jnp.zeros((16,), jnp.float32)
    wbase = wid * BCAP

    def z_acc(i, _):
        acc[pl.ds(i * 16, 16)] = zf16
        return 0
    lax.fori_loop(0, (RPW + 1) * D // 16, z_acc, 0)

    pltpu.sync_copy(cnt_hbm.at[pl.ds(wid * 16, 16)], cbuf)
    cnt = cbuf[pl.ds(0, 16)][0]
    nb = cnt // BQ

    def stage(b, db, sb, semd, sems):
        base = wbase + b * BQ
        pltpu.make_async_copy(bdst_hbm.at[pl.ds(base, BQ)], db, semd).start()
        pltpu.make_async_copy(bsrc_hbm.at[pl.ds(base, BQ)], sb, sems).start()

    def wait(b, db, sb, semd, sems):
        base = wbase + b * BQ
        pltpu.make_async_copy(bdst_hbm.at[pl.ds(base, BQ)], db, semd).wait()
        pltpu.make_async_copy(bsrc_hbm.at[pl.ds(base, BQ)], sb, sems).wait()

    def gather_blk(sb, b, rbuf, sem):
        return pltpu.make_async_copy(
            m_hbm.at[sb.at[pl.ds(b * KB, KB)]], rbuf, sem)

    def process_blk(db, b, rbuf):
        for g in range(KB // 16):
            vd = db[pl.ds(b * KB + g * 16, 16)]
            for l in range(16):
                rb = vd[l] * D
                for j in range(8):
                    sl = pl.ds(rb + j * 16, 16)
                    acc[sl] = jnp.maximum(
                        acc[sl], rbuf[g * 16 + l, pl.ds(j * 16, 16)])

    NBLK = BQ // KB

    def process_bq(db, sb):
        gather_blk(sb, 0, rows0, sg0).start()

        def pair_body(p, _):
            b0 = p * 2
            b1 = b0 + 1
            gather_blk(sb, b1, rows1, sg1).start()
            gather_blk(sb, b0, rows0, sg0).wait()
            process_blk(db, b0, rows0)

            @pl.when(b0 + 2 < NBLK)
            def _():
                gather_blk(sb, b0 + 2, rows0, sg0).start()

            gather_blk(sb, b1, rows1, sg1).wait()
            process_blk(db, b1, rows1)
            return 0
        lax.fori_loop(0, NBLK // 2, pair_body, 0)

    @pl.when(nb > 0)
    def _():
        stage(0, d0, s0, sed0, ses0)

    def blk_body(b, _):
        even = b % 2 == 0

        @pl.when(even)
        def _():
            @pl.when(b + 1 < nb)
            def _():
                stage(b + 1, d1, s1, sed1, ses1)
            wait(b, d0, s0, sed0, ses0)
            process_bq(d0, s0)

        @pl.when(jnp.logical_not(even))
        def _():
            @pl.when(b + 1 < nb)
            def _():
                stage(b + 1, d0, s0, sed0, ses0)
            wait(b, d1, s1, sed1, ses1)
            process_bq(d1, s1)
        return 0
    lax.fori_loop(0, nb, blk_body, 0)

    @pl.when(wid < NW - 1)
    def _():
        pltpu.sync_copy(acc.at[pl.ds(0, RPW * D)],
                        pooled_hbm.at[pl.ds(lo * D, RPW * D)])

    @pl.when(wid == NW - 1)
    def _():
        pltpu.sync_copy(acc.at[pl.ds(0, LAST * D)],
                        pooled_hbm.at[pl.ds(lo * D, LAST * D)])


_gathermax = pl.kernel(
    _gathermax_body,
    out_type=jax.ShapeDtypeStruct((N * D,), jnp.float32),
    mesh=plsc.VectorSubcoreMesh(core_axis_name="c", subcore_axis_name="s"),
    scratch_types=[
        pltpu.VMEM((BQ,), jnp.int32),
        pltpu.VMEM((BQ,), jnp.int32),
        pltpu.VMEM((BQ,), jnp.int32),
        pltpu.VMEM((BQ,), jnp.int32),
        pltpu.VMEM((KB, D), jnp.float32),
        pltpu.VMEM((KB, D), jnp.float32),
        pltpu.VMEM(((RPW + 1) * D,), jnp.float32),
        pltpu.VMEM((16,), jnp.int32),
        pltpu.SemaphoreType.DMA,
        pltpu.SemaphoreType.DMA,
        pltpu.SemaphoreType.DMA,
        pltpu.SemaphoreType.DMA,
        pltpu.SemaphoreType.DMA,
        pltpu.SemaphoreType.DMA,
    ],
    compiler_params=pltpu.CompilerParams(needs_layout_passes=False),
)


def _dot_t(x, w):
    return lax.dot_general(x, w, (((1,), (1,)), ((), ())),
                           preferred_element_type=jnp.float32)


def _pre_body(h_ref, w_ref, b_ref, o_ref):
    m = _dot_t(h_ref[...], w_ref[...]) + b_ref[...]
    o_ref[...] = jnp.maximum(m, 0.0)


def _pre(h, w, b):
    return pl.pallas_call(
        _pre_body,
        out_shape=jax.ShapeDtypeStruct((h.shape[0], w.shape[0]), jnp.float32),
    )(h, w, b.reshape(1, -1))


def _post_body(h_ref, p_ref, ws_ref, wn_ref, b_ref, o_ref):
    t = _dot_t(h_ref[...], ws_ref[...]) + _dot_t(p_ref[...], wn_ref[...])
    t = t + b_ref[...]
    nrm = jnp.sqrt(jnp.sum(t * t, axis=1, keepdims=True))
    t = t / jnp.maximum(nrm, 1e-12)
    o_ref[...] = jnp.maximum(t, 0.0)


def _post(h, pooled, ws, wn, b):
    return pl.pallas_call(
        _post_body,
        out_shape=jax.ShapeDtypeStruct((h.shape[0], ws.shape[0]), jnp.float32),
    )(h, pooled, ws, wn, b.reshape(1, -1))


def kernel(inputs, edge_index, W_pool1, b_pool1, W_self1, W_neigh1, bias1,
           W_pool2, b_pool2, W_self2, W_neigh2, bias2):
    src = edge_index[0]
    dst = edge_index[1]
    bdst, bsrc, cnts = _partition(src, dst)
    m1 = _pre(inputs, W_pool1, b_pool1)
    pooled1 = _gathermax(m1, bdst, bsrc, cnts).reshape(N, D)
    h1 = _post(inputs, pooled1, W_self1, W_neigh1, bias1)
    m2 = _pre(h1, W_pool2, b_pool2)
    pooled2 = _gathermax(m2, bdst, bsrc, cnts).reshape(N, D)
    return _post(h1, pooled2, W_self2, W_neigh2, bias2)


# speedup vs baseline: 3.1632x; 1.7948x over previous
import jax
import jax.numpy as jnp
from jax import lax
from jax.experimental import pallas as pl
from jax.experimental.pallas import tpu as pltpu
from jax.experimental.pallas import tpu_sc as plsc

N = 10000
E = 320000
D = 128

NC, NS = 2, 16
NW = NC * NS
RPW = 320
LAST = N - (NW - 1) * RPW
CHUNK = 8000
NCHUNK = E // CHUNK
KB = 16
MBUF = CHUNK + KB + 16
TRASH = CHUNK + KB
DUMMY = RPW
FB = 8192
FCAP = 16384
BQ = 512
BCAP = E + NCHUNK * 16 + BQ
BCAP = (BCAP + BQ - 1) // BQ * BQ


def _partition_body(src_hbm, dst_hbm, bdst_hbm, bsrc_hbm, cnt_hbm,
                    dst0, src0, dst1, src1, mdst, msrc, fdst, fsrc, cbuf,
                    sd0, ss0, sd1, ss1, sf0, sf1):
    c = lax.axis_index("c")
    s = lax.axis_index("s")
    wid = s * NC + c
    lo = wid * RPW
    zi16 = jnp.zeros((16,), jnp.int32)
    onev = jnp.ones((16,), jnp.int32)
    dummyv = jnp.full((16,), DUMMY, jnp.int32)
    trashv = jnp.full((16,), TRASH, jnp.int32)
    lov = lax.broadcast(lo, (16,))
    hiv = lax.broadcast(lo + RPW, (16,))
    wbase = wid * BCAP

    def stage(ci, db, sb, semd, sems):
        base = ci * CHUNK
        cpd = pltpu.make_async_copy(dst_hbm.at[pl.ds(base, CHUNK)], db, semd)
        cps = pltpu.make_async_copy(src_hbm.at[pl.ds(base, CHUNK)], sb, sems)
        cpd.start()
        cps.start()
        return cpd, cps

    def process_chunk(db, sb, carry):
        fill, off = carry
        def scan_body(i, cntv):
            vd = db[pl.ds(i * 16, 16)]
            vs = sb[pl.ds(i * 16, 16)]
            msk = (vd >= lov) & (vd < hiv)
            cs = plsc.cumsum(jnp.where(msk, onev, zi16))
            pos = jnp.where(msk, cntv + cs - onev, trashv)
            plsc.store_scatter(mdst, [pos], vd - lov)
            plsc.store_scatter(msrc, [pos], vs)
            n = plsc.all_reduce_population_count(msk)
            return cntv + n
        cntv = lax.fori_loop(0, CHUNK // 16, scan_body, zi16)
        cnt = cntv[0]

        mdst[pl.ds(cnt, 16)] = dummyv
        msrc[pl.ds(cnt, 16)] = zi16
        cnt16 = (cnt + 15) // 16 * 16

        def app_body(i, _):
            fdst[pl.ds(fill + i * 16, 16)] = mdst[pl.ds(i * 16, 16)]
            fsrc[pl.ds(fill + i * 16, 16)] = msrc[pl.ds(i * 16, 16)]
            return 0
        lax.fori_loop(0, cnt16 // 16, app_body, 0)
        nfill = fill + cnt16

        @pl.when(nfill >= FB)
        def _():
            dpos = wbase + off * FB
            pltpu.sync_copy(fdst.at[pl.ds(0, FB)], bdst_hbm.at[pl.ds(dpos, FB)])
            pltpu.sync_copy(fsrc.at[pl.ds(0, FB)], bsrc_hbm.at[pl.ds(dpos, FB)])
            rem = nfill - FB

            def mv_body(i, _):
                fdst[pl.ds(i * 16, 16)] = fdst[pl.ds(FB + i * 16, 16)]
                fsrc[pl.ds(i * 16, 16)] = fsrc[pl.ds(FB + i * 16, 16)]
                return 0
            lax.fori_loop(0, (rem + 15) // 16, mv_body, 0)

        flushed = nfill >= FB
        return (jnp.where(flushed, nfill - FB, nfill),
                jnp.where(flushed, off + 1, off))

    stage(0, dst0, src0, sd0, ss0)

    def cpair_body(p, carry):
        c0 = p * 2
        c1 = c0 + 1
        stage(c1, dst1, src1, sd1, ss1)
        pltpu.make_async_copy(dst_hbm.at[pl.ds(c0 * CHUNK, CHUNK)], dst0, sd0).wait()
        pltpu.make_async_copy(src_hbm.at[pl.ds(c0 * CHUNK, CHUNK)], src0, ss0).wait()
        carry = process_chunk(dst0, src0, carry)

        @pl.when(c0 + 2 < NCHUNK)
        def _():
            stage(c0 + 2, dst0, src0, sd0, ss0)

        pltpu.make_async_copy(dst_hbm.at[pl.ds(c1 * CHUNK, CHUNK)], dst1, sd1).wait()
        pltpu.make_async_copy(src_hbm.at[pl.ds(c1 * CHUNK, CHUNK)], src1, ss1).wait()
        return process_chunk(dst1, src1, carry)

    fill, off = lax.fori_loop(0, NCHUNK // 2, cpair_body,
                              (jnp.int32(0), jnp.int32(0)))

    padv = (BQ - fill % BQ) % BQ

    def pad_body(i, _):
        fdst[pl.ds(fill + i * 16, 16)] = dummyv
        fsrc[pl.ds(fill + i * 16, 16)] = zi16
        return 0
    lax.fori_loop(0, padv // 16, pad_body, 0)
    fill = fill + padv

    def fin_body(j, _):
        dpos = wbase + off * FB + j * BQ
        pltpu.sync_copy(fdst.at[pl.ds(j * BQ, BQ)], bdst_hbm.at[pl.ds(dpos, BQ)])
        pltpu.sync_copy(fsrc.at[pl.ds(j * BQ, BQ)], bsrc_hbm.at[pl.ds(dpos, BQ)])
        return 0
    lax.fori_loop(0, fill // BQ, fin_body, 0)

    total = off * FB + fill
    cbuf[pl.ds(0, 16)] = lax.broadcast(total, (16,))
    pltpu.sync_copy(cbuf, cnt_hbm.at[pl.ds(wid * 16, 16)])


_partition = pl.kernel(
    _partition_body,
    out_type=(jax.ShapeDtypeStruct((NW * BCAP,), jnp.int32),
              jax.ShapeDtypeStruct((NW * BCAP,), jnp.int32),
              jax.ShapeDtypeStruct((NW * 16,), jnp.int32)),
    mesh=plsc.VectorSubcoreMesh(core_axis_name="c", subcore_axis_name="s"),
    scratch_types=[
        pltpu.VMEM((CHUNK,), jnp.int32),
        pltpu.VMEM((CHUNK,), jnp.int32),
        pltpu.VMEM((CHUNK,), jnp.int32),
        pltpu.VMEM((CHUNK,), jnp.int32),
        pltpu.VMEM((MBUF,), jnp.int32),
        pltpu.VMEM((MBUF,), jnp.int32),
        pltpu.VMEM((FCAP,), jnp.int32),
        pltpu.VMEM((FCAP,), jnp.int32),
        pltpu.VMEM((16,), jnp.int32),
        pltpu.SemaphoreType.DMA,
        pltpu.SemaphoreType.DMA,
        pltpu.SemaphoreType.DMA,
        pltpu.SemaphoreType.DMA,
        pltpu.SemaphoreType.DMA,
        pltpu.SemaphoreType.DMA,
    ],
    compiler_params=pltpu.CompilerParams(needs_layout_passes=False),
)


MROWS = 624
MLAST = N - (NS - 1) * MROWS


def _gathermax_body(m_hbm, bdst_hbm, bsrc_hbm, cnt_hbm, pooled_hbm,
                    d0, s0, d1, s1, rows0, rows1, acc, cbuf, mshr,
                    sed0, ses0, sed1, ses1, sg0, sg1):
    c = lax.axis_index("c")
    s = lax.axis_index("s")
    wid = s * NC + c
    lo = wid * RPW
    zf16 = jnp.zeros((16,), jnp.float32)
    wbase = wid * BCAP

    @pl.when(s < NS - 1)
    def _():
        pltpu.sync_copy(m_hbm.at[pl.ds(s * MROWS, MROWS)],
                        mshr.at[pl.ds(s * MROWS, MROWS)])

    @pl.when(s == NS - 1)
    def _():
        pltpu.sync_copy(m_hbm.at[pl.ds(s * MROWS, MLAST)],
                        mshr.at[pl.ds(s * MROWS, MLAST)])

    def z_acc(i, _):
        acc[pl.ds(i * 16, 16)] = zf16
        return 0
    lax.fori_loop(0, (RPW + 1) * D // 16, z_acc, 0)

    plsc.subcore_barrier()

    pltpu.sync_copy(cnt_hbm.at[pl.ds(wid * 16, 16)], cbuf)
    cnt = cbuf[pl.ds(0, 16)][0]
    nb = cnt // BQ

    def stage(b, db, sb, semd, sems):
        base = wbase + b * BQ
        pltpu.make_async_copy(bdst_hbm.at[pl.ds(base, BQ)], db, semd).start()
        pltpu.make_async_copy(bsrc_hbm.at[pl.ds(base, BQ)], sb, sems).start()

    def wait(b, db, sb, semd, sems):
        base = wbase + b * BQ
        pltpu.make_async_copy(bdst_hbm.at[pl.ds(base, BQ)], db, semd).wait()
        pltpu.make_async_copy(bsrc_hbm.at[pl.ds(base, BQ)], sb, sems).wait()

    def gather_blk(sb, b, rbuf, sem):
        return pltpu.make_async_copy(
            mshr.at[sb.at[pl.ds(b * KB, KB)]], rbuf, sem)

    def process_blk(db, b, rbuf):
        for g in range(KB // 16):
            vd = db[pl.ds(b * KB + g * 16, 16)]
            for l in range(16):
                rb = vd[l] * D
                for j in range(8):
                    sl = pl.ds(rb + j * 16, 16)
                    acc[sl] = jnp.maximum(
                        acc[sl], rbuf[g * 16 + l, pl.ds(j * 16, 16)])

    NBLK = BQ // KB

    def process_bq(db, sb):
        gather_blk(sb, 0, rows0, sg0).start()

        def pair_body(p, _):
            b0 = p * 2
            b1 = b0 + 1
            gather_blk(sb, b1, rows1, sg1).start()
            gather_blk(sb, b0, rows0, sg0).wait()
            process_blk(db, b0, rows0)

            @pl.when(b0 + 2 < NBLK)
            def _():
                gather_blk(sb, b0 + 2, rows0, sg0).start()

            gather_blk(sb, b1, rows1, sg1).wait()
            process_blk(db, b1, rows1)
            return 0
        lax.fori_loop(0, NBLK // 2, pair_body, 0)

    @pl.when(nb > 0)
    def _():
        stage(0, d0, s0, sed0, ses0)

    def blk_body(b, _):
        even = b % 2 == 0

        @pl.when(even)
        def _():
            @pl.when(b + 1 < nb)
            def _():
                stage(b + 1, d1, s1, sed1, ses1)
            wait(b, d0, s0, sed0, ses0)
            process_bq(d0, s0)

        @pl.when(jnp.logical_not(even))
        def _():
            @pl.when(b + 1 < nb)
            def _():
                stage(b + 1, d0, s0, sed0, ses0)
            wait(b, d1, s1, sed1, ses1)
            process_bq(d1, s1)
        return 0
    lax.fori_loop(0, nb, blk_body, 0)

    @pl.when(wid < NW - 1)
    def _():
        pltpu.sync_copy(acc.at[pl.ds(0, RPW * D)],
                        pooled_hbm.at[pl.ds(lo * D, RPW * D)])

    @pl.when(wid == NW - 1)
    def _():
        pltpu.sync_copy(acc.at[pl.ds(0, LAST * D)],
                        pooled_hbm.at[pl.ds(lo * D, LAST * D)])


_gathermax = pl.kernel(
    _gathermax_body,
    out_type=jax.ShapeDtypeStruct((N * D,), jnp.float32),
    mesh=plsc.VectorSubcoreMesh(core_axis_name="c", subcore_axis_name="s"),
    scratch_types=[
        pltpu.VMEM((BQ,), jnp.int32),
        pltpu.VMEM((BQ,), jnp.int32),
        pltpu.VMEM((BQ,), jnp.int32),
        pltpu.VMEM((BQ,), jnp.int32),
        pltpu.VMEM((KB, D), jnp.float32),
        pltpu.VMEM((KB, D), jnp.float32),
        pltpu.VMEM(((RPW + 1) * D,), jnp.float32),
        pltpu.VMEM((16,), jnp.int32),
        pltpu.VMEM_SHARED((N, D), jnp.float32),
        pltpu.SemaphoreType.DMA,
        pltpu.SemaphoreType.DMA,
        pltpu.SemaphoreType.DMA,
        pltpu.SemaphoreType.DMA,
        pltpu.SemaphoreType.DMA,
        pltpu.SemaphoreType.DMA,
    ],
    compiler_params=pltpu.CompilerParams(needs_layout_passes=False),
)


def _dot_t(x, w):
    return lax.dot_general(x, w, (((1,), (1,)), ((), ())),
                           preferred_element_type=jnp.float32)


def _pre_body(h_ref, w_ref, b_ref, o_ref):
    m = _dot_t(h_ref[...], w_ref[...]) + b_ref[...]
    o_ref[...] = jnp.maximum(m, 0.0)


def _pre(h, w, b):
    return pl.pallas_call(
        _pre_body,
        out_shape=jax.ShapeDtypeStruct((h.shape[0], w.shape[0]), jnp.float32),
    )(h, w, b.reshape(1, -1))


def _post_body(h_ref, p_ref, ws_ref, wn_ref, b_ref, o_ref):
    t = _dot_t(h_ref[...], ws_ref[...]) + _dot_t(p_ref[...], wn_ref[...])
    t = t + b_ref[...]
    nrm = jnp.sqrt(jnp.sum(t * t, axis=1, keepdims=True))
    t = t / jnp.maximum(nrm, 1e-12)
    o_ref[...] = jnp.maximum(t, 0.0)


def _post(h, pooled, ws, wn, b):
    return pl.pallas_call(
        _post_body,
        out_shape=jax.ShapeDtypeStruct((h.shape[0], ws.shape[0]), jnp.float32),
    )(h, pooled, ws, wn, b.reshape(1, -1))


def kernel(inputs, edge_index, W_pool1, b_pool1, W_self1, W_neigh1, bias1,
           W_pool2, b_pool2, W_self2, W_neigh2, bias2):
    src = edge_index[0]
    dst = edge_index[1]
    bdst, bsrc, cnts = _partition(src, dst)
    m1 = _pre(inputs, W_pool1, b_pool1)
    pooled1 = _gathermax(m1, bdst, bsrc, cnts).reshape(N, D)
    h1 = _post(inputs, pooled1, W_self1, W_neigh1, bias1)
    m2 = _pre(h1, W_pool2, b_pool2)
    pooled2 = _gathermax(m2, bdst, bsrc, cnts).reshape(N, D)
    return _post(h1, pooled2, W_self2, W_neigh2, bias2)


# speedup vs baseline: 4.1267x; 1.3046x over previous
import jax
import jax.numpy as jnp
from jax import lax
from jax.experimental import pallas as pl
from jax.experimental.pallas import tpu as pltpu
from jax.experimental.pallas import tpu_sc as plsc

N = 10000
E = 320000
D = 128

NC, NS = 2, 16
NW = NC * NS
RPW = 320
LAST = N - (NW - 1) * RPW
CHUNK = 8000
NCHUNK = E // CHUNK
KB = 32
MBUF = CHUNK + KB + 16
TRASH = CHUNK + KB
DUMMY = RPW
FB = 8192
FCAP = 16384
BQ = 512
BCAP = E + NCHUNK * 16 + BQ
BCAP = (BCAP + BQ - 1) // BQ * BQ


def _partition_body(src_hbm, dst_hbm, bdst_hbm, bsrc_hbm, cnt_hbm,
                    dst0, src0, dst1, src1, mdst, msrc, fdst, fsrc, cbuf,
                    sd0, ss0, sd1, ss1, sf0, sf1):
    c = lax.axis_index("c")
    s = lax.axis_index("s")
    wid = s * NC + c
    lo = wid * RPW
    zi16 = jnp.zeros((16,), jnp.int32)
    onev = jnp.ones((16,), jnp.int32)
    dummyv = jnp.full((16,), DUMMY, jnp.int32)
    trashv = jnp.full((16,), TRASH, jnp.int32)
    lov = lax.broadcast(lo, (16,))
    hiv = lax.broadcast(lo + RPW, (16,))
    wbase = wid * BCAP

    def stage(ci, db, sb, semd, sems):
        base = ci * CHUNK
        cpd = pltpu.make_async_copy(dst_hbm.at[pl.ds(base, CHUNK)], db, semd)
        cps = pltpu.make_async_copy(src_hbm.at[pl.ds(base, CHUNK)], sb, sems)
        cpd.start()
        cps.start()
        return cpd, cps

    def process_chunk(db, sb, carry):
        fill, off = carry
        def scan_body(i, cntv):
            vd = db[pl.ds(i * 16, 16)]
            vs = sb[pl.ds(i * 16, 16)]
            msk = (vd >= lov) & (vd < hiv)
            cs = plsc.cumsum(jnp.where(msk, onev, zi16))
            pos = jnp.where(msk, cntv + cs - onev, trashv)
            plsc.store_scatter(mdst, [pos], vd - lov)
            plsc.store_scatter(msrc, [pos], vs)
            n = plsc.all_reduce_population_count(msk)
            return cntv + n
        cntv = lax.fori_loop(0, CHUNK // 16, scan_body, zi16)
        cnt = cntv[0]

        mdst[pl.ds(cnt, 16)] = dummyv
        msrc[pl.ds(cnt, 16)] = zi16
        cnt16 = (cnt + 15) // 16 * 16

        def app_body(i, _):
            fdst[pl.ds(fill + i * 16, 16)] = mdst[pl.ds(i * 16, 16)]
            fsrc[pl.ds(fill + i * 16, 16)] = msrc[pl.ds(i * 16, 16)]
            return 0
        lax.fori_loop(0, cnt16 // 16, app_body, 0)
        nfill = fill + cnt16

        @pl.when(nfill >= FB)
        def _():
            dpos = wbase + off * FB
            pltpu.sync_copy(fdst.at[pl.ds(0, FB)], bdst_hbm.at[pl.ds(dpos, FB)])
            pltpu.sync_copy(fsrc.at[pl.ds(0, FB)], bsrc_hbm.at[pl.ds(dpos, FB)])
            rem = nfill - FB

            def mv_body(i, _):
                fdst[pl.ds(i * 16, 16)] = fdst[pl.ds(FB + i * 16, 16)]
                fsrc[pl.ds(i * 16, 16)] = fsrc[pl.ds(FB + i * 16, 16)]
                return 0
            lax.fori_loop(0, (rem + 15) // 16, mv_body, 0)

        flushed = nfill >= FB
        return (jnp.where(flushed, nfill - FB, nfill),
                jnp.where(flushed, off + 1, off))

    stage(0, dst0, src0, sd0, ss0)

    def cpair_body(p, carry):
        c0 = p * 2
        c1 = c0 + 1
        stage(c1, dst1, src1, sd1, ss1)
        pltpu.make_async_copy(dst_hbm.at[pl.ds(c0 * CHUNK, CHUNK)], dst0, sd0).wait()
        pltpu.make_async_copy(src_hbm.at[pl.ds(c0 * CHUNK, CHUNK)], src0, ss0).wait()
        carry = process_chunk(dst0, src0, carry)

        @pl.when(c0 + 2 < NCHUNK)
        def _():
            stage(c0 + 2, dst0, src0, sd0, ss0)

        pltpu.make_async_copy(dst_hbm.at[pl.ds(c1 * CHUNK, CHUNK)], dst1, sd1).wait()
        pltpu.make_async_copy(src_hbm.at[pl.ds(c1 * CHUNK, CHUNK)], src1, ss1).wait()
        return process_chunk(dst1, src1, carry)

    fill, off = lax.fori_loop(0, NCHUNK // 2, cpair_body,
                              (jnp.int32(0), jnp.int32(0)))

    padv = (BQ - fill % BQ) % BQ

    def pad_body(i, _):
        fdst[pl.ds(fill + i * 16, 16)] = dummyv
        fsrc[pl.ds(fill + i * 16, 16)] = zi16
        return 0
    lax.fori_loop(0, padv // 16, pad_body, 0)
    fill = fill + padv

    def fin_body(j, _):
        dpos = wbase + off * FB + j * BQ
        pltpu.sync_copy(fdst.at[pl.ds(j * BQ, BQ)], bdst_hbm.at[pl.ds(dpos, BQ)])
        pltpu.sync_copy(fsrc.at[pl.ds(j * BQ, BQ)], bsrc_hbm.at[pl.ds(dpos, BQ)])
        return 0
    lax.fori_loop(0, fill // BQ, fin_body, 0)

    total = off * FB + fill
    cbuf[pl.ds(0, 16)] = lax.broadcast(total, (16,))
    pltpu.sync_copy(cbuf, cnt_hbm.at[pl.ds(wid * 16, 16)])


_partition = pl.kernel(
    _partition_body,
    out_type=(jax.ShapeDtypeStruct((NW * BCAP,), jnp.int32),
              jax.ShapeDtypeStruct((NW * BCAP,), jnp.int32),
              jax.ShapeDtypeStruct((NW * 16,), jnp.int32)),
    mesh=plsc.VectorSubcoreMesh(core_axis_name="c", subcore_axis_name="s"),
    scratch_types=[
        pltpu.VMEM((CHUNK,), jnp.int32),
        pltpu.VMEM((CHUNK,), jnp.int32),
        pltpu.VMEM((CHUNK,), jnp.int32),
        pltpu.VMEM((CHUNK,), jnp.int32),
        pltpu.VMEM((MBUF,), jnp.int32),
        pltpu.VMEM((MBUF,), jnp.int32),
        pltpu.VMEM((FCAP,), jnp.int32),
        pltpu.VMEM((FCAP,), jnp.int32),
        pltpu.VMEM((16,), jnp.int32),
        pltpu.SemaphoreType.DMA,
        pltpu.SemaphoreType.DMA,
        pltpu.SemaphoreType.DMA,
        pltpu.SemaphoreType.DMA,
        pltpu.SemaphoreType.DMA,
        pltpu.SemaphoreType.DMA,
    ],
    compiler_params=pltpu.CompilerParams(needs_layout_passes=False),
)


MROWS = 624
MLAST = N - (NS - 1) * MROWS
DH = D // 2


def _gathermax_body(m_hbm, bdst_hbm, bsrc_hbm, cnt_hbm, pooled_hbm,
                    d0, s0, d1, s1, rows0, rows1, acc, cbuf, mshr,
                    sed0, ses0, sed1, ses1, sg0, sg1):
    c = lax.axis_index("c")
    s = lax.axis_index("s")
    wid = s * NC + c
    lo = wid * RPW
    zi16 = jnp.zeros((16,), jnp.int32)
    wbase = wid * BCAP

    @pl.when(s < NS - 1)
    def _():
        pltpu.sync_copy(m_hbm.at[pl.ds(s * MROWS, MROWS)],
                        mshr.at[pl.ds(s * MROWS, MROWS)])

    @pl.when(s == NS - 1)
    def _():
        pltpu.sync_copy(m_hbm.at[pl.ds(s * MROWS, MLAST)],
                        mshr.at[pl.ds(s * MROWS, MLAST)])

    def z_acc(i, _):
        acc[pl.ds(i * 16, 16)] = zi16
        return 0
    lax.fori_loop(0, (RPW + 1) * DH // 16, z_acc, 0)

    plsc.subcore_barrier()

    pltpu.sync_copy(cnt_hbm.at[pl.ds(wid * 16, 16)], cbuf)
    cnt = cbuf[pl.ds(0, 16)][0]
    nb = cnt // BQ

    def stage(b, db, sb, semd, sems):
        base = wbase + b * BQ
        pltpu.make_async_copy(bdst_hbm.at[pl.ds(base, BQ)], db, semd).start()
        pltpu.make_async_copy(bsrc_hbm.at[pl.ds(base, BQ)], sb, sems).start()

    def wait(b, db, sb, semd, sems):
        base = wbase + b * BQ
        pltpu.make_async_copy(bdst_hbm.at[pl.ds(base, BQ)], db, semd).wait()
        pltpu.make_async_copy(bsrc_hbm.at[pl.ds(base, BQ)], sb, sems).wait()

    def gather_blk(sb, b, rbuf, sem):
        return pltpu.make_async_copy(
            mshr.at[sb.at[pl.ds(b * KB, KB)]], rbuf, sem)

    def process_blk(db, b, rbuf):
        for g in range(KB // 16):
            vd = db[pl.ds(b * KB + g * 16, 16)]
            for l in range(16):
                rb = vd[l] * DH
                for j in range(4):
                    sl = pl.ds(rb + j * 16, 16)
                    a = plsc.bitcast(acc[sl], jnp.bfloat16)
                    r = plsc.bitcast(rbuf[g * 16 + l, pl.ds(j * 16, 16)],
                                     jnp.bfloat16)
                    acc[sl] = plsc.bitcast(jnp.maximum(a, r), jnp.int32)

    NBLK = BQ // KB

    def process_bq(db, sb):
        gather_blk(sb, 0, rows0, sg0).start()

        def pair_body(p, _):
            b0 = p * 2
            b1 = b0 + 1
            gather_blk(sb, b1, rows1, sg1).start()
            gather_blk(sb, b0, rows0, sg0).wait()
            process_blk(db, b0, rows0)

            @pl.when(b0 + 2 < NBLK)
            def _():
                gather_blk(sb, b0 + 2, rows0, sg0).start()

            gather_blk(sb, b1, rows1, sg1).wait()
            process_blk(db, b1, rows1)
            return 0
        lax.fori_loop(0, NBLK // 2, pair_body, 0)

    @pl.when(nb > 0)
    def _():
        stage(0, d0, s0, sed0, ses0)

    def blk_body(b, _):
        even = b % 2 == 0

        @pl.when(even)
        def _():
            @pl.when(b + 1 < nb)
            def _():
                stage(b + 1, d1, s1, sed1, ses1)
            wait(b, d0, s0, sed0, ses0)
            process_bq(d0, s0)

        @pl.when(jnp.logical_not(even))
        def _():
            @pl.when(b + 1 < nb)
            def _():
                stage(b + 1, d0, s0, sed0, ses0)
            wait(b, d1, s1, sed1, ses1)
            process_bq(d1, s1)
        return 0
    lax.fori_loop(0, nb, blk_body, 0)

    @pl.when(wid < NW - 1)
    def _():
        pltpu.sync_copy(acc.at[pl.ds(0, RPW * DH)],
                        pooled_hbm.at[pl.ds(lo * DH, RPW * DH)])

    @pl.when(wid == NW - 1)
    def _():
        pltpu.sync_copy(acc.at[pl.ds(0, LAST * DH)],
                        pooled_hbm.at[pl.ds(lo * DH, LAST * DH)])


_gathermax = pl.kernel(
    _gathermax_body,
    out_type=jax.ShapeDtypeStruct((N * DH,), jnp.int32),
    mesh=plsc.VectorSubcoreMesh(core_axis_name="c", subcore_axis_name="s"),
    scratch_types=[
        pltpu.VMEM((BQ,), jnp.int32),
        pltpu.VMEM((BQ,), jnp.int32),
        pltpu.VMEM((BQ,), jnp.int32),
        pltpu.VMEM((BQ,), jnp.int32),
        pltpu.VMEM((KB, D), jnp.int32),
        pltpu.VMEM((KB, D), jnp.int32),
        pltpu.VMEM(((RPW + 1) * DH,), jnp.int32),
        pltpu.VMEM((16,), jnp.int32),
        pltpu.VMEM_SHARED((N, D), jnp.int32),
        pltpu.SemaphoreType.DMA,
        pltpu.SemaphoreType.DMA,
        pltpu.SemaphoreType.DMA,
        pltpu.SemaphoreType.DMA,
        pltpu.SemaphoreType.DMA,
        pltpu.SemaphoreType.DMA,
    ],
    compiler_params=pltpu.CompilerParams(needs_layout_passes=False),
)


def _dot_t(x, w):
    return lax.dot_general(x, w, (((1,), (1,)), ((), ())),
                           preferred_element_type=jnp.float32)


def _pre_body(h_ref, w_ref, b_ref, o_ref):
    m = _dot_t(h_ref[...], w_ref[...]) + b_ref[...]
    o_ref[...] = jnp.maximum(m, 0.0).astype(jnp.bfloat16)


def _pre(h, w, b):
    return pl.pallas_call(
        _pre_body,
        out_shape=jax.ShapeDtypeStruct((h.shape[0], w.shape[0]), jnp.bfloat16),
    )(h, w, b.reshape(1, -1))


def _post_body(h_ref, p_ref, ws_ref, wn_ref, b_ref, o_ref):
    p = p_ref[...].astype(jnp.float32)
    t = _dot_t(h_ref[...], ws_ref[...]) + _dot_t(p, wn_ref[...])
    t = t + b_ref[...]
    nrm = jnp.sqrt(jnp.sum(t * t, axis=1, keepdims=True))
    t = t / jnp.maximum(nrm, 1e-12)
    o_ref[...] = jnp.maximum(t, 0.0)


def _post(h, pooled, ws, wn, b):
    return pl.pallas_call(
        _post_body,
        out_shape=jax.ShapeDtypeStruct((h.shape[0], ws.shape[0]), jnp.float32),
    )(h, pooled, ws, wn, b.reshape(1, -1))


def _to_packed(m):
    p = lax.bitcast_convert_type(m.reshape(N, DH, 2), jnp.int32)
    return jnp.concatenate([p, jnp.zeros((N, D - DH), jnp.int32)], axis=1)


def _from_packed(p):
    return lax.bitcast_convert_type(
        p.reshape(N, DH), jnp.bfloat16).reshape(N, D)


def kernel(inputs, edge_index, W_pool1, b_pool1, W_self1, W_neigh1, bias1,
           W_pool2, b_pool2, W_self2, W_neigh2, bias2):
    src = edge_index[0]
    dst = edge_index[1]
    bdst, bsrc, cnts = _partition(src, dst)
    m1 = _to_packed(_pre(inputs, W_pool1, b_pool1))
    pooled1 = _from_packed(_gathermax(m1, bdst, bsrc, cnts))
    h1 = _post(inputs, pooled1, W_self1, W_neigh1, bias1)
    m2 = _to_packed(_pre(h1, W_pool2, b_pool2))
    pooled2 = _from_packed(_gathermax(m2, bdst, bsrc, cnts))
    return _post(h1, pooled2, W_self2, W_neigh2, bias2)


# speedup vs baseline: 4.1376x; 1.0026x over previous
import jax
import jax.numpy as jnp
from jax import lax
from jax.experimental import pallas as pl
from jax.experimental.pallas import tpu as pltpu
from jax.experimental.pallas import tpu_sc as plsc

N = 10000
E = 320000
D = 128

NC, NS = 2, 16
NW = NC * NS
RPW = 320
LAST = N - (NW - 1) * RPW
CHUNK = 8000
NCHUNK = E // CHUNK
KB = 32
MBUF = CHUNK + KB + 16
TRASH = CHUNK + KB
DUMMY = RPW
FB = 8192
FCAP = 16384
BQ = 512
BCAP = E + NCHUNK * 16 + BQ
BCAP = (BCAP + BQ - 1) // BQ * BQ


SHIFT = 16384


def _partition_body(edge_hbm, bedg_hbm, cnt_hbm,
                    e0, e1, medg, fedg, cbuf,
                    se0, se1, sf0):
    c = lax.axis_index("c")
    s = lax.axis_index("s")
    wid = s * NC + c
    lo = wid * RPW
    zi16 = jnp.zeros((16,), jnp.int32)
    onev = jnp.ones((16,), jnp.int32)
    dummyv = jnp.full((16,), DUMMY * SHIFT, jnp.int32)
    trashv = jnp.full((16,), TRASH, jnp.int32)
    lov = lax.broadcast(lo * SHIFT, (16,))
    hiv = lax.broadcast((lo + RPW) * SHIFT, (16,))
    wbase = wid * BCAP

    def stage(ci, eb, sem):
        pltpu.make_async_copy(
            edge_hbm.at[pl.ds(ci * CHUNK, CHUNK)], eb, sem).start()

    def process_chunk(eb, carry):
        fill, off = carry
        def scan_body(i, cntv):
            v = eb[pl.ds(i * 16, 16)]
            msk = (v >= lov) & (v < hiv)
            cs = plsc.cumsum(jnp.where(msk, onev, zi16))
            pos = jnp.where(msk, cntv + cs - onev, trashv)
            plsc.store_scatter(medg, [pos], v - lov)
            n = plsc.all_reduce_population_count(msk)
            return cntv + n
        cntv = lax.fori_loop(0, CHUNK // 16, scan_body, zi16)
        cnt = cntv[0]

        medg[pl.ds(cnt, 16)] = dummyv
        cnt16 = (cnt + 15) // 16 * 16

        def app_body(i, _):
            fedg[pl.ds(fill + i * 16, 16)] = medg[pl.ds(i * 16, 16)]
            return 0
        lax.fori_loop(0, cnt16 // 16, app_body, 0)
        nfill = fill + cnt16

        @pl.when(nfill >= FB)
        def _():
            dpos = wbase + off * FB
            pltpu.sync_copy(fedg.at[pl.ds(0, FB)], bedg_hbm.at[pl.ds(dpos, FB)])
            rem = nfill - FB

            def mv_body(i, _):
                fedg[pl.ds(i * 16, 16)] = fedg[pl.ds(FB + i * 16, 16)]
                return 0
            lax.fori_loop(0, (rem + 15) // 16, mv_body, 0)

        flushed = nfill >= FB
        return (jnp.where(flushed, nfill - FB, nfill),
                jnp.where(flushed, off + 1, off))

    stage(0, e0, se0)

    def cpair_body(p, carry):
        c0 = p * 2
        c1 = c0 + 1
        stage(c1, e1, se1)
        pltpu.make_async_copy(edge_hbm.at[pl.ds(c0 * CHUNK, CHUNK)], e0, se0).wait()
        carry = process_chunk(e0, carry)

        @pl.when(c0 + 2 < NCHUNK)
        def _():
            stage(c0 + 2, e0, se0)

        pltpu.make_async_copy(edge_hbm.at[pl.ds(c1 * CHUNK, CHUNK)], e1, se1).wait()
        return process_chunk(e1, carry)

    fill, off = lax.fori_loop(0, NCHUNK // 2, cpair_body,
                              (jnp.int32(0), jnp.int32(0)))

    padv = (BQ - fill % BQ) % BQ

    def pad_body(i, _):
        fedg[pl.ds(fill + i * 16, 16)] = dummyv
        return 0
    lax.fori_loop(0, padv // 16, pad_body, 0)
    fill = fill + padv

    def fin_body(j, _):
        dpos = wbase + off * FB + j * BQ
        pltpu.sync_copy(fedg.at[pl.ds(j * BQ, BQ)], bedg_hbm.at[pl.ds(dpos, BQ)])
        return 0
    lax.fori_loop(0, fill // BQ, fin_body, 0)

    total = off * FB + fill
    cbuf[pl.ds(0, 16)] = lax.broadcast(total, (16,))
    pltpu.sync_copy(cbuf, cnt_hbm.at[pl.ds(wid * 16, 16)])


_partition = pl.kernel(
    _partition_body,
    out_type=(jax.ShapeDtypeStruct((NW * BCAP,), jnp.int32),
              jax.ShapeDtypeStruct((NW * 16,), jnp.int32)),
    mesh=plsc.VectorSubcoreMesh(core_axis_name="c", subcore_axis_name="s"),
    scratch_types=[
        pltpu.VMEM((CHUNK,), jnp.int32),
        pltpu.VMEM((CHUNK,), jnp.int32),
        pltpu.VMEM((MBUF,), jnp.int32),
        pltpu.VMEM((FCAP,), jnp.int32),
        pltpu.VMEM((16,), jnp.int32),
        pltpu.SemaphoreType.DMA,
        pltpu.SemaphoreType.DMA,
        pltpu.SemaphoreType.DMA,
    ],
    compiler_params=pltpu.CompilerParams(needs_layout_passes=False),
)


MROWS = 624
MLAST = N - (NS - 1) * MROWS
DH = D // 2


def _gathermax_body(m_hbm, bedg_hbm, cnt_hbm, pooled_hbm,
                    e0, e1, sbuf, rows0, rows1, acc, cbuf, mshr,
                    see0, see1, sg0, sg1):
    c = lax.axis_index("c")
    s = lax.axis_index("s")
    wid = s * NC + c
    lo = wid * RPW
    zi16 = jnp.zeros((16,), jnp.int32)
    wbase = wid * BCAP

    @pl.when(s < NS - 1)
    def _():
        pltpu.sync_copy(m_hbm.at[pl.ds(s * MROWS, MROWS)],
                        mshr.at[pl.ds(s * MROWS, MROWS)])

    @pl.when(s == NS - 1)
    def _():
        pltpu.sync_copy(m_hbm.at[pl.ds(s * MROWS, MLAST)],
                        mshr.at[pl.ds(s * MROWS, MLAST)])

    def z_acc(i, _):
        acc[pl.ds(i * 16, 16)] = zi16
        return 0
    lax.fori_loop(0, (RPW + 1) * DH // 16, z_acc, 0)

    plsc.subcore_barrier()

    pltpu.sync_copy(cnt_hbm.at[pl.ds(wid * 16, 16)], cbuf)
    cnt = cbuf[pl.ds(0, 16)][0]
    nb = cnt // BQ

    def stage(b, eb, sem):
        pltpu.make_async_copy(
            bedg_hbm.at[pl.ds(wbase + b * BQ, BQ)], eb, sem).start()

    def wait(b, eb, sem):
        pltpu.make_async_copy(
            bedg_hbm.at[pl.ds(wbase + b * BQ, BQ)], eb, sem).wait()

    def gather_blk(b, rbuf, sem):
        return pltpu.make_async_copy(
            mshr.at[sbuf.at[pl.ds(b * KB, KB)]], rbuf, sem)

    def process_blk(eb, b, rbuf):
        for g in range(KB // 16):
            vd = lax.shift_right_logical(eb[pl.ds(b * KB + g * 16, 16)], 14)
            for l in range(16):
                rb = vd[l] * DH
                for j in range(4):
                    sl = pl.ds(rb + j * 16, 16)
                    a = plsc.bitcast(acc[sl], jnp.bfloat16)
                    r = plsc.bitcast(rbuf[g * 16 + l, pl.ds(j * 16, 16)],
                                     jnp.bfloat16)
                    acc[sl] = plsc.bitcast(jnp.maximum(a, r), jnp.int32)

    NBLK = BQ // KB

    def process_bq(eb):
        def unp_body(i, _):
            sbuf[pl.ds(i * 16, 16)] = jnp.bitwise_and(
                eb[pl.ds(i * 16, 16)], SHIFT - 1)
            return 0
        lax.fori_loop(0, BQ // 16, unp_body, 0)

        gather_blk(0, rows0, sg0).start()

        def pair_body(p, _):
            b0 = p * 2
            b1 = b0 + 1
            gather_blk(b1, rows1, sg1).start()
            gather_blk(b0, rows0, sg0).wait()
            process_blk(eb, b0, rows0)

            @pl.when(b0 + 2 < NBLK)
            def _():
                gather_blk(b0 + 2, rows0, sg0).start()

            gather_blk(b1, rows1, sg1).wait()
            process_blk(eb, b1, rows1)
            return 0
        lax.fori_loop(0, NBLK // 2, pair_body, 0)

    @pl.when(nb > 0)
    def _():
        stage(0, e0, see0)

    def blk_body(b, _):
        even = b % 2 == 0

        @pl.when(even)
        def _():
            @pl.when(b + 1 < nb)
            def _():
                stage(b + 1, e1, see1)
            wait(b, e0, see0)
            process_bq(e0)

        @pl.when(jnp.logical_not(even))
        def _():
            @pl.when(b + 1 < nb)
            def _():
                stage(b + 1, e0, see0)
            wait(b, e1, see1)
            process_bq(e1)
        return 0
    lax.fori_loop(0, nb, blk_body, 0)

    @pl.when(wid < NW - 1)
    def _():
        pltpu.sync_copy(acc.at[pl.ds(0, RPW * DH)],
                        pooled_hbm.at[pl.ds(lo * DH, RPW * DH)])

    @pl.when(wid == NW - 1)
    def _():
        pltpu.sync_copy(acc.at[pl.ds(0, LAST * DH)],
                        pooled_hbm.at[pl.ds(lo * DH, LAST * DH)])


_gathermax = pl.kernel(
    _gathermax_body,
    out_type=jax.ShapeDtypeStruct((N * DH,), jnp.int32),
    mesh=plsc.VectorSubcoreMesh(core_axis_name="c", subcore_axis_name="s"),
    scratch_types=[
        pltpu.VMEM((BQ,), jnp.int32),
        pltpu.VMEM((BQ,), jnp.int32),
        pltpu.VMEM((BQ,), jnp.int32),
        pltpu.VMEM((KB, D), jnp.int32),
        pltpu.VMEM((KB, D), jnp.int32),
        pltpu.VMEM(((RPW + 1) * DH,), jnp.int32),
        pltpu.VMEM((16,), jnp.int32),
        pltpu.VMEM_SHARED((N, D), jnp.int32),
        pltpu.SemaphoreType.DMA,
        pltpu.SemaphoreType.DMA,
        pltpu.SemaphoreType.DMA,
        pltpu.SemaphoreType.DMA,
    ],
    compiler_params=pltpu.CompilerParams(needs_layout_passes=False),
)


def _dot_t(x, w):
    return lax.dot_general(x, w, (((1,), (1,)), ((), ())),
                           preferred_element_type=jnp.float32)


def _pre_body(h_ref, w_ref, b_ref, o_ref):
    m = _dot_t(h_ref[...], w_ref[...]) + b_ref[...]
    o_ref[...] = jnp.maximum(m, 0.0).astype(jnp.bfloat16)


def _pre(h, w, b):
    return pl.pallas_call(
        _pre_body,
        out_shape=jax.ShapeDtypeStruct((h.shape[0], w.shape[0]), jnp.bfloat16),
    )(h, w, b.reshape(1, -1))


def _post_body(h_ref, p_ref, ws_ref, wn_ref, b_ref, o_ref):
    p = p_ref[...].astype(jnp.float32)
    t = _dot_t(h_ref[...], ws_ref[...]) + _dot_t(p, wn_ref[...])
    t = t + b_ref[...]
    nrm = jnp.sqrt(jnp.sum(t * t, axis=1, keepdims=True))
    t = t / jnp.maximum(nrm, 1e-12)
    o_ref[...] = jnp.maximum(t, 0.0)


def _post(h, pooled, ws, wn, b):
    return pl.pallas_call(
        _post_body,
        out_shape=jax.ShapeDtypeStruct((h.shape[0], ws.shape[0]), jnp.float32),
    )(h, pooled, ws, wn, b.reshape(1, -1))


def _to_packed(m):
    p = lax.bitcast_convert_type(m.reshape(N, DH, 2), jnp.int32)
    return jnp.concatenate([p, jnp.zeros((N, D - DH), jnp.int32)], axis=1)


def _from_packed(p):
    return lax.bitcast_convert_type(
        p.reshape(N, DH), jnp.bfloat16).reshape(N, D)


def kernel(inputs, edge_index, W_pool1, b_pool1, W_self1, W_neigh1, bias1,
           W_pool2, b_pool2, W_self2, W_neigh2, bias2):
    packed = edge_index[1] * SHIFT + edge_index[0]
    bedg, cnts = _partition(packed)
    m1 = _to_packed(_pre(inputs, W_pool1, b_pool1))
    pooled1 = _from_packed(_gathermax(m1, bedg, cnts))
    h1 = _post(inputs, pooled1, W_self1, W_neigh1, bias1)
    m2 = _to_packed(_pre(h1, W_pool2, b_pool2))
    pooled2 = _from_packed(_gathermax(m2, bedg, cnts))
    return _post(h1, pooled2, W_self2, W_neigh2, bias2)


# speedup vs baseline: 5.6679x; 1.3699x over previous
import jax
import jax.numpy as jnp
from jax import lax
from jax.experimental import pallas as pl
from jax.experimental.pallas import tpu as pltpu
from jax.experimental.pallas import tpu_sc as plsc

N = 10000
E = 320000
D = 128

NC, NS = 2, 16
NW = NC * NS
RPW = 320
LAST = N - (NW - 1) * RPW
CHUNK = 8000
NCHUNK = E // CHUNK
KB = 32
MBUF = CHUNK + KB + 16
TRASH = CHUNK + KB
DUMMY = RPW
FB = 8192
FCAP = 16384
BQ = 512
BCAP = E + NCHUNK * 16 + BQ
BCAP = (BCAP + BQ - 1) // BQ * BQ


SHIFT = 16384


def _partition_body(edge_hbm, bedg_hbm, cnt_hbm,
                    e0, e1, medg, fedg, cbuf,
                    se0, se1, sf0):
    c = lax.axis_index("c")
    s = lax.axis_index("s")
    wid = s * NC + c
    lo = wid * RPW
    zi16 = jnp.zeros((16,), jnp.int32)
    onev = jnp.ones((16,), jnp.int32)
    dummyv = jnp.full((16,), DUMMY * SHIFT, jnp.int32)
    trashv = jnp.full((16,), TRASH, jnp.int32)
    lov = lax.broadcast(lo * SHIFT, (16,))
    hiv = lax.broadcast((lo + RPW) * SHIFT, (16,))
    wbase = wid * BCAP

    def stage(ci, eb, sem):
        pltpu.make_async_copy(
            edge_hbm.at[pl.ds(ci * CHUNK, CHUNK)], eb, sem).start()

    def process_chunk(eb, carry):
        fill, off = carry
        def scan_body(i, cntv):
            v = eb[pl.ds(i * 16, 16)]
            msk = (v >= lov) & (v < hiv)
            cs = plsc.cumsum(jnp.where(msk, onev, zi16))
            pos = jnp.where(msk, cntv + cs - onev, trashv)
            plsc.store_scatter(medg, [pos], v - lov)
            n = plsc.all_reduce_population_count(msk)
            return cntv + n
        cntv = plsc.parallel_loop(0, CHUNK // 16, unroll=4,
                                  carry=zi16)(scan_body)
        cnt = cntv[0]

        medg[pl.ds(cnt, 16)] = dummyv
        cnt16 = (cnt + 15) // 16 * 16

        def app_body(i):
            fedg[pl.ds(fill + i * 16, 16)] = medg[pl.ds(i * 16, 16)]
        plsc.parallel_loop(0, cnt16 // 16, unroll=2)(app_body)
        nfill = fill + cnt16

        @pl.when(nfill >= FB)
        def _():
            dpos = wbase + off * FB
            pltpu.sync_copy(fedg.at[pl.ds(0, FB)], bedg_hbm.at[pl.ds(dpos, FB)])
            rem = nfill - FB

            def mv_body(i):
                fedg[pl.ds(i * 16, 16)] = fedg[pl.ds(FB + i * 16, 16)]
            plsc.parallel_loop(0, (rem + 15) // 16, unroll=2)(mv_body)

        flushed = nfill >= FB
        return (jnp.where(flushed, nfill - FB, nfill),
                jnp.where(flushed, off + 1, off))

    stage(0, e0, se0)

    def cpair_body(p, carry):
        c0 = p * 2
        c1 = c0 + 1
        stage(c1, e1, se1)
        pltpu.make_async_copy(edge_hbm.at[pl.ds(c0 * CHUNK, CHUNK)], e0, se0).wait()
        carry = process_chunk(e0, carry)

        @pl.when(c0 + 2 < NCHUNK)
        def _():
            stage(c0 + 2, e0, se0)

        pltpu.make_async_copy(edge_hbm.at[pl.ds(c1 * CHUNK, CHUNK)], e1, se1).wait()
        return process_chunk(e1, carry)

    fill, off = lax.fori_loop(0, NCHUNK // 2, cpair_body,
                              (jnp.int32(0), jnp.int32(0)))

    padv = (BQ - fill % BQ) % BQ

    def pad_body(i, _):
        fedg[pl.ds(fill + i * 16, 16)] = dummyv
        return 0
    lax.fori_loop(0, padv // 16, pad_body, 0)
    fill = fill + padv

    def fin_body(j, _):
        dpos = wbase + off * FB + j * BQ
        pltpu.sync_copy(fedg.at[pl.ds(j * BQ, BQ)], bedg_hbm.at[pl.ds(dpos, BQ)])
        return 0
    lax.fori_loop(0, fill // BQ, fin_body, 0)

    total = off * FB + fill
    cbuf[pl.ds(0, 16)] = lax.broadcast(total, (16,))
    pltpu.sync_copy(cbuf, cnt_hbm.at[pl.ds(wid * 16, 16)])


_partition = pl.kernel(
    _partition_body,
    out_type=(jax.ShapeDtypeStruct((NW * BCAP,), jnp.int32),
              jax.ShapeDtypeStruct((NW * 16,), jnp.int32)),
    mesh=plsc.VectorSubcoreMesh(core_axis_name="c", subcore_axis_name="s"),
    scratch_types=[
        pltpu.VMEM((CHUNK,), jnp.int32),
        pltpu.VMEM((CHUNK,), jnp.int32),
        pltpu.VMEM((MBUF,), jnp.int32),
        pltpu.VMEM((FCAP,), jnp.int32),
        pltpu.VMEM((16,), jnp.int32),
        pltpu.SemaphoreType.DMA,
        pltpu.SemaphoreType.DMA,
        pltpu.SemaphoreType.DMA,
    ],
    compiler_params=pltpu.CompilerParams(needs_layout_passes=False),
)


MROWS = 624
MLAST = N - (NS - 1) * MROWS
DH = D // 2


def _gathermax_body(m_hbm, bedg_hbm, cnt_hbm, pooled_hbm,
                    e0, e1, sbuf, rows0, rows1, acc, cbuf, mshr,
                    see0, see1, sg0, sg1):
    c = lax.axis_index("c")
    s = lax.axis_index("s")
    wid = s * NC + c
    lo = wid * RPW
    zi16 = jnp.zeros((16,), jnp.int32)
    wbase = wid * BCAP

    @pl.when(s < NS - 1)
    def _():
        pltpu.sync_copy(m_hbm.at[pl.ds(s * MROWS, MROWS)],
                        mshr.at[pl.ds(s * MROWS, MROWS)])

    @pl.when(s == NS - 1)
    def _():
        pltpu.sync_copy(m_hbm.at[pl.ds(s * MROWS, MLAST)],
                        mshr.at[pl.ds(s * MROWS, MLAST)])

    def z_acc(i):
        acc[pl.ds(i * 16, 16)] = zi16
    plsc.parallel_loop(0, (RPW + 1) * DH // 16, unroll=4)(z_acc)

    plsc.subcore_barrier()

    pltpu.sync_copy(cnt_hbm.at[pl.ds(wid * 16, 16)], cbuf)
    cnt = cbuf[pl.ds(0, 16)][0]
    nb = cnt // BQ

    def stage(b, eb, sem):
        pltpu.make_async_copy(
            bedg_hbm.at[pl.ds(wbase + b * BQ, BQ)], eb, sem).start()

    def wait(b, eb, sem):
        pltpu.make_async_copy(
            bedg_hbm.at[pl.ds(wbase + b * BQ, BQ)], eb, sem).wait()

    def gather_blk(b, rbuf, sem):
        return pltpu.make_async_copy(
            mshr.at[sbuf.at[pl.ds(b * KB, KB)]], rbuf, sem)

    def process_blk(eb, b, rbuf):
        for g in range(KB // 16):
            vd = lax.shift_right_logical(eb[pl.ds(b * KB + g * 16, 16)], 14)
            for l in range(16):
                rb = vd[l] * DH
                for j in range(4):
                    sl = pl.ds(rb + j * 16, 16)
                    a = plsc.bitcast(acc[sl], jnp.bfloat16)
                    r = plsc.bitcast(rbuf[g * 16 + l, pl.ds(j * 16, 16)],
                                     jnp.bfloat16)
                    acc[sl] = plsc.bitcast(jnp.maximum(a, r), jnp.int32)

    NBLK = BQ // KB

    def process_bq(eb):
        def unp_body(i):
            sbuf[pl.ds(i * 16, 16)] = jnp.bitwise_and(
                eb[pl.ds(i * 16, 16)], SHIFT - 1)
        plsc.parallel_loop(0, BQ // 16, unroll=4)(unp_body)

        gather_blk(0, rows0, sg0).start()

        def pair_body(p, _):
            b0 = p * 2
            b1 = b0 + 1
            gather_blk(b1, rows1, sg1).start()
            gather_blk(b0, rows0, sg0).wait()
            process_blk(eb, b0, rows0)

            @pl.when(b0 + 2 < NBLK)
            def _():
                gather_blk(b0 + 2, rows0, sg0).start()

            gather_blk(b1, rows1, sg1).wait()
            process_blk(eb, b1, rows1)
            return 0
        lax.fori_loop(0, NBLK // 2, pair_body, 0)

    @pl.when(nb > 0)
    def _():
        stage(0, e0, see0)

    def blk_body(b, _):
        even = b % 2 == 0

        @pl.when(even)
        def _():
            @pl.when(b + 1 < nb)
            def _():
                stage(b + 1, e1, see1)
            wait(b, e0, see0)
            process_bq(e0)

        @pl.when(jnp.logical_not(even))
        def _():
            @pl.when(b + 1 < nb)
            def _():
                stage(b + 1, e0, see0)
            wait(b, e1, see1)
            process_bq(e1)
        return 0
    lax.fori_loop(0, nb, blk_body, 0)

    @pl.when(wid < NW - 1)
    def _():
        pltpu.sync_copy(acc.at[pl.ds(0, RPW * DH)],
                        pooled_hbm.at[pl.ds(lo * DH, RPW * DH)])

    @pl.when(wid == NW - 1)
    def _():
        pltpu.sync_copy(acc.at[pl.ds(0, LAST * DH)],
                        pooled_hbm.at[pl.ds(lo * DH, LAST * DH)])


_gathermax = pl.kernel(
    _gathermax_body,
    out_type=jax.ShapeDtypeStruct((N * DH,), jnp.int32),
    mesh=plsc.VectorSubcoreMesh(core_axis_name="c", subcore_axis_name="s"),
    scratch_types=[
        pltpu.VMEM((BQ,), jnp.int32),
        pltpu.VMEM((BQ,), jnp.int32),
        pltpu.VMEM((BQ,), jnp.int32),
        pltpu.VMEM((KB, D), jnp.int32),
        pltpu.VMEM((KB, D), jnp.int32),
        pltpu.VMEM(((RPW + 1) * DH,), jnp.int32),
        pltpu.VMEM((16,), jnp.int32),
        pltpu.VMEM_SHARED((N, D), jnp.int32),
        pltpu.SemaphoreType.DMA,
        pltpu.SemaphoreType.DMA,
        pltpu.SemaphoreType.DMA,
        pltpu.SemaphoreType.DMA,
    ],
    compiler_params=pltpu.CompilerParams(needs_layout_passes=False),
)


def _dot_t(x, w):
    return lax.dot_general(x, w, (((1,), (1,)), ((), ())),
                           preferred_element_type=jnp.float32)


def _pre_body(h_ref, w_ref, b_ref, o_ref):
    m = _dot_t(h_ref[...], w_ref[...]) + b_ref[...]
    o_ref[...] = jnp.maximum(m, 0.0).astype(jnp.bfloat16)


def _pre(h, w, b):
    return pl.pallas_call(
        _pre_body,
        out_shape=jax.ShapeDtypeStruct((h.shape[0], w.shape[0]), jnp.bfloat16),
    )(h, w, b.reshape(1, -1))


def _post_body(h_ref, p_ref, ws_ref, wn_ref, b_ref, o_ref):
    p = p_ref[...].astype(jnp.float32)
    t = _dot_t(h_ref[...], ws_ref[...]) + _dot_t(p, wn_ref[...])
    t = t + b_ref[...]
    nrm = jnp.sqrt(jnp.sum(t * t, axis=1, keepdims=True))
    t = t / jnp.maximum(nrm, 1e-12)
    o_ref[...] = jnp.maximum(t, 0.0)


def _post(h, pooled, ws, wn, b):
    return pl.pallas_call(
        _post_body,
        out_shape=jax.ShapeDtypeStruct((h.shape[0], ws.shape[0]), jnp.float32),
    )(h, pooled, ws, wn, b.reshape(1, -1))


def _to_packed(m):
    p = lax.bitcast_convert_type(m.reshape(N, DH, 2), jnp.int32)
    return jnp.concatenate([p, jnp.zeros((N, D - DH), jnp.int32)], axis=1)


def _from_packed(p):
    return lax.bitcast_convert_type(
        p.reshape(N, DH), jnp.bfloat16).reshape(N, D)


def kernel(inputs, edge_index, W_pool1, b_pool1, W_self1, W_neigh1, bias1,
           W_pool2, b_pool2, W_self2, W_neigh2, bias2):
    packed = edge_index[1] * SHIFT + edge_index[0]
    bedg, cnts = _partition(packed)
    m1 = _to_packed(_pre(inputs, W_pool1, b_pool1))
    pooled1 = _from_packed(_gathermax(m1, bedg, cnts))
    h1 = _post(inputs, pooled1, W_self1, W_neigh1, bias1)
    m2 = _to_packed(_pre(h1, W_pool2, b_pool2))
    pooled2 = _from_packed(_gathermax(m2, bedg, cnts))
    return _post(h1, pooled2, W_self2, W_neigh2, bias2)
